# baseline (speedup 1.0000x reference)
import functools

import jax
import jax.numpy as jnp
from jax import lax
from jax.experimental import pallas as pl
from jax.experimental.pallas import tpu as pltpu
from jax.experimental.pallas import tpu_sc as plsc


NBUF = 4


@functools.partial(jax.jit, static_argnames=("n", "d", "nw", "b_per_w", "c"))
def _gather_rows(idx, table, n, d, nw, b_per_w, c):
    n_chunks = b_per_w // c
    mesh = plsc.VectorSubcoreMesh(core_axis_name="c", subcore_axis_name="s")

    @functools.partial(
        pl.kernel,
        mesh=mesh,
        out_type=jax.ShapeDtypeStruct((n, d), jnp.float32),
        scratch_types=[
            pltpu.VMEM((b_per_w,), jnp.int32),
            pltpu.VMEM((NBUF, c, d), jnp.float32),
            pltpu.SemaphoreType.DMA((NBUF,)),
            pltpu.SemaphoreType.DMA((NBUF,)),
        ],
    )
    def k(idx_hbm, table_hbm, out_hbm, idx_v, rows_v, gsem, ssem):
        nc = plsc.get_sparse_core_info().num_cores
        wid = lax.axis_index("s") * nc + lax.axis_index("c")
        base = wid * b_per_w
        s_len = idx_hbm.shape[1]
        w_per_row = s_len // b_per_w
        pltpu.sync_copy(
            idx_hbm.at[wid // w_per_row,
                       pl.ds(lax.rem(wid, w_per_row) * b_per_w, b_per_w)],
            idx_v,
        )

        for b in range(NBUF):
            pltpu.async_copy(
                table_hbm.at[idx_v.at[pl.ds(b * c, c)]], rows_v.at[b], gsem.at[b]
            )

        def body(i, _):
            b = lax.rem(i, NBUF)
            pltpu.make_async_copy(
                table_hbm.at[pl.ds(0, c)], rows_v.at[b], gsem.at[b]
            ).wait()
            store = pltpu.async_copy(
                rows_v.at[b], out_hbm.at[pl.ds(base + i * c, c)], ssem.at[b]
            )

            @pl.when(i + NBUF < n_chunks)
            def _():
                store.wait()
                pltpu.async_copy(
                    table_hbm.at[idx_v.at[pl.ds((i + NBUF) * c, c)]],
                    rows_v.at[b],
                    gsem.at[b],
                )

            return None

        lax.fori_loop(0, n_chunks, body, None)
        for b in range(NBUF):
            pltpu.make_async_copy(
                table_hbm.at[pl.ds(0, c)], rows_v.at[b], ssem.at[b]
            ).wait()

    return k(idx, table)


def kernel(tokens, W_E):
    b, s = tokens.shape
    v, d = W_E.shape
    n = b * s
    info = plsc.get_sparse_core_info()
    nw = info.num_cores * info.num_subcores
    assert n % nw == 0
    b_per_w = n // nw
    c = 32
    while b_per_w % c:
        c //= 2
    out = _gather_rows(tokens, W_E, n, d, nw, b_per_w, c)
    return out.reshape(b, s, d)


# speedup vs baseline: 1.0063x; 1.0063x over previous
import functools

import jax
import jax.numpy as jnp
from jax import lax
from jax.experimental import pallas as pl
from jax.experimental.pallas import tpu as pltpu
from jax.experimental.pallas import tpu_sc as plsc


NBUF = 4


@functools.partial(jax.jit, static_argnames=("n", "d", "nw", "b_per_w", "c"))
def _gather_rows(idx, table, n, d, nw, b_per_w, c):
    n_chunks = b_per_w // c
    mesh = plsc.VectorSubcoreMesh(core_axis_name="c", subcore_axis_name="s")

    @functools.partial(
        pl.kernel,
        mesh=mesh,
        out_type=jax.ShapeDtypeStruct((n, d), jnp.float32),
        scratch_types=[
            pltpu.VMEM((b_per_w,), jnp.int32),
            pltpu.VMEM((NBUF, c, d), jnp.float32),
            pltpu.SemaphoreType.DMA((NBUF,)),
            pltpu.SemaphoreType.DMA((NBUF,)),
        ],
    )
    def k(idx_hbm, table_hbm, out_hbm, idx_v, rows_v, gsem, ssem):
        nc = plsc.get_sparse_core_info().num_cores
        wid = lax.axis_index("s") * nc + lax.axis_index("c")
        base = wid * b_per_w
        s_len = idx_hbm.shape[1]
        w_per_row = s_len // b_per_w
        pltpu.sync_copy(
            idx_hbm.at[wid // w_per_row,
                       pl.ds(lax.rem(wid, w_per_row) * b_per_w, b_per_w)],
            idx_v,
        )

        for b in range(NBUF):
            pltpu.async_copy(
                table_hbm.at[idx_v.at[pl.ds(b * c, c)]], rows_v.at[b], gsem.at[b]
            )

        def body(i, _):
            b = lax.rem(i, NBUF)
            pltpu.make_async_copy(
                table_hbm.at[pl.ds(0, c)], rows_v.at[b], gsem.at[b]
            ).wait()
            store = pltpu.async_copy(
                rows_v.at[b], out_hbm.at[pl.ds(base + i * c, c)], ssem.at[b]
            )

            @pl.when(i + NBUF < n_chunks)
            def _():
                store.wait()
                pltpu.async_copy(
                    table_hbm.at[idx_v.at[pl.ds((i + NBUF) * c, c)]],
                    rows_v.at[b],
                    gsem.at[b],
                )

            return None

        lax.fori_loop(0, n_chunks, body, None)
        for b in range(NBUF):
            pltpu.make_async_copy(
                table_hbm.at[pl.ds(0, c)], rows_v.at[b], ssem.at[b]
            ).wait()

    return k(idx, table)


def kernel(tokens, W_E):
    b, s = tokens.shape
    v, d = W_E.shape
    n = b * s
    info = plsc.get_sparse_core_info()
    nw = info.num_cores * info.num_subcores
    assert n % nw == 0
    b_per_w = n // nw
    c = 32
    while b_per_w % c:
        c //= 2
    out = _gather_rows(tokens.astype(jnp.int32), W_E, n, d, nw, b_per_w, c)
    return out.reshape(b, s, d)


# speedup vs baseline: 1.0114x; 1.0051x over previous
import functools

import jax
import jax.numpy as jnp
from jax import lax
from jax.experimental import pallas as pl
from jax.experimental.pallas import tpu as pltpu
from jax.experimental.pallas import tpu_sc as plsc


NBUF = 5


@functools.partial(jax.jit, static_argnames=("n", "d", "nw", "b_per_w", "c"))
def _gather_rows(idx, table, n, d, nw, b_per_w, c):
    n_chunks = b_per_w // c
    mesh = plsc.VectorSubcoreMesh(core_axis_name="c", subcore_axis_name="s")

    @functools.partial(
        pl.kernel,
        mesh=mesh,
        out_type=jax.ShapeDtypeStruct((n, d), jnp.float32),
        scratch_types=[
            pltpu.VMEM((b_per_w,), jnp.int32),
            pltpu.VMEM((NBUF, c, d), jnp.float32),
            pltpu.SemaphoreType.DMA((NBUF,)),
            pltpu.SemaphoreType.DMA((NBUF,)),
        ],
    )
    def k(idx_hbm, table_hbm, out_hbm, idx_v, rows_v, gsem, ssem):
        nc = plsc.get_sparse_core_info().num_cores
        wid = lax.axis_index("s") * nc + lax.axis_index("c")
        base = wid * b_per_w
        s_len = idx_hbm.shape[1]
        w_per_row = s_len // b_per_w
        pltpu.sync_copy(
            idx_hbm.at[wid // w_per_row,
                       pl.ds(lax.rem(wid, w_per_row) * b_per_w, b_per_w)],
            idx_v,
        )

        for b in range(NBUF):
            pltpu.async_copy(
                table_hbm.at[idx_v.at[pl.ds(b * c, c)]], rows_v.at[b], gsem.at[b]
            )

        def body(i, _):
            b = lax.rem(i, NBUF)
            pltpu.make_async_copy(
                table_hbm.at[pl.ds(0, c)], rows_v.at[b], gsem.at[b]
            ).wait()
            store = pltpu.async_copy(
                rows_v.at[b], out_hbm.at[pl.ds(base + i * c, c)], ssem.at[b]
            )

            @pl.when(i + NBUF < n_chunks)
            def _():
                store.wait()
                pltpu.async_copy(
                    table_hbm.at[idx_v.at[pl.ds((i + NBUF) * c, c)]],
                    rows_v.at[b],
                    gsem.at[b],
                )

            return None

        lax.fori_loop(0, n_chunks, body, None)
        for b in range(NBUF):
            pltpu.make_async_copy(
                table_hbm.at[pl.ds(0, c)], rows_v.at[b], ssem.at[b]
            ).wait()

    return k(idx, table)


def kernel(tokens, W_E):
    b, s = tokens.shape
    v, d = W_E.shape
    n = b * s
    info = plsc.get_sparse_core_info()
    nw = info.num_cores * info.num_subcores
    assert n % nw == 0
    b_per_w = n // nw
    c = 32
    while b_per_w % c:
        c //= 2
    out = _gather_rows(tokens.astype(jnp.int32), W_E, n, d, nw, b_per_w, c)
    return out.reshape(b, s, d)
